# baseline (speedup 1.0000x reference)
import jax
import jax.numpy as jnp
import numpy as np
from jax.experimental import pallas as pl

_NUM_SAMPLES = 16


def _sample_indices(t: int) -> np.ndarray:
    stop = np.float32(t - 1)
    frac = np.arange(_NUM_SAMPLES - 1, dtype=np.float32) / np.float32(
        _NUM_SAMPLES - 1
    )
    vals = np.concatenate([stop * frac, np.array([stop], np.float32)])
    vals = np.clip(vals, np.float32(0.0), stop)
    return vals.astype(np.int32)


def kernel(x):
    t, c, hh, ww = x.shape
    p_total = c * hh * ww
    src = _sample_indices(t)

    sel_np = np.zeros((_NUM_SAMPLES, t), np.float32)
    sel_np[np.arange(_NUM_SAMPLES), src] = 1.0
    sel = jnp.asarray(sel_np)

    rows_per_block = 8
    bn = rows_per_block * ww
    grid = p_total // bn
    assert p_total % bn == 0 and hh % rows_per_block == 0
    hblocks = hh // rows_per_block

    def body(sel_ref, x_ref, o_ref):
        res = jax.lax.dot_general(
            sel_ref[...],
            x_ref[...],
            (((1,), (1,)), ((), ())),
            preferred_element_type=jnp.float32,
            precision=jax.lax.Precision.HIGHEST,
        )
        o_ref[...] = res.reshape(_NUM_SAMPLES, 1, rows_per_block, ww)

    out = pl.pallas_call(
        body,
        grid=(grid,),
        in_specs=[
            pl.BlockSpec((_NUM_SAMPLES, t), lambda n: (0, 0)),
            pl.BlockSpec((bn, t), lambda n: (n, 0)),
        ],
        out_specs=pl.BlockSpec(
            (_NUM_SAMPLES, 1, rows_per_block, ww),
            lambda n: (0, n // hblocks, n % hblocks, 0),
        ),
        out_shape=jax.ShapeDtypeStruct((_NUM_SAMPLES, c, hh, ww), jnp.float32),
    )(sel, x.transpose(1, 2, 3, 0).reshape(p_total, t))

    return out


# speedup vs baseline: 1.3492x; 1.3492x over previous
import jax
import jax.numpy as jnp
import numpy as np
from jax.experimental import pallas as pl

_NUM_SAMPLES = 16


def _sample_indices(t: int) -> np.ndarray:
    stop = np.float32(t - 1)
    frac = np.arange(_NUM_SAMPLES - 1, dtype=np.float32) / np.float32(
        _NUM_SAMPLES - 1
    )
    vals = np.concatenate([stop * frac, np.array([stop], np.float32)])
    vals = np.clip(vals, np.float32(0.0), stop)
    return vals.astype(np.int32)


def kernel(x):
    t, c, hh, ww = x.shape
    p_total = c * hh * ww
    src = _sample_indices(t)

    sel_np = np.zeros((_NUM_SAMPLES, t), np.float32)
    sel_np[np.arange(_NUM_SAMPLES), src] = 1.0
    sel = jnp.asarray(sel_np)

    rows_per_block = 8
    bn = rows_per_block * ww
    grid = p_total // bn
    assert p_total % bn == 0 and hh % rows_per_block == 0
    hblocks = hh // rows_per_block

    def body(sel_ref, x_ref, o_ref):
        res = jax.lax.dot_general(
            sel_ref[...],
            x_ref[...],
            (((1,), (1,)), ((), ())),
            preferred_element_type=jnp.float32,

        )
        o_ref[...] = res.reshape(_NUM_SAMPLES, 1, rows_per_block, ww)

    out = pl.pallas_call(
        body,
        grid=(grid,),
        in_specs=[
            pl.BlockSpec((_NUM_SAMPLES, t), lambda n: (0, 0)),
            pl.BlockSpec((bn, t), lambda n: (n, 0)),
        ],
        out_specs=pl.BlockSpec(
            (_NUM_SAMPLES, 1, rows_per_block, ww),
            lambda n: (0, n // hblocks, n % hblocks, 0),
        ),
        out_shape=jax.ShapeDtypeStruct((_NUM_SAMPLES, c, hh, ww), jnp.float32),
    )(sel, x.transpose(1, 2, 3, 0).reshape(p_total, t))

    return out


# speedup vs baseline: 1.9575x; 1.4509x over previous
import functools

import jax
import jax.numpy as jnp
import numpy as np
from jax.experimental import pallas as pl
from jax.experimental.pallas import tpu as pltpu

_NUM_SAMPLES = 16


def _sample_indices(t: int) -> np.ndarray:
    stop = np.float32(t - 1)
    frac = np.arange(_NUM_SAMPLES - 1, dtype=np.float32) / np.float32(
        _NUM_SAMPLES - 1
    )
    vals = np.concatenate([stop * frac, np.array([stop], np.float32)])
    vals = np.clip(vals, np.float32(0.0), stop)
    return vals.astype(np.int32)


def kernel(x):
    t, c, hh, ww = x.shape
    p_total = c * hh * ww
    src = _sample_indices(t)

    sel_np = np.zeros((_NUM_SAMPLES, t), np.float32)
    sel_np[np.arange(_NUM_SAMPLES), src] = 1.0
    sel = jnp.asarray(sel_np)

    bn = 7168
    grid = p_total // bn
    assert p_total % bn == 0

    def body(sel_ref, x_ref, o_ref):
        o_ref[...] = jax.lax.dot_general(
            sel_ref[...],
            x_ref[...],
            ((( 1,), (1,)), ((), ())),
            preferred_element_type=jnp.float32,
        )

    out = pl.pallas_call(
        body,
        grid=(grid,),
        in_specs=[
            pl.BlockSpec((_NUM_SAMPLES, t), lambda n: (0, 0)),
            pl.BlockSpec((bn, t), lambda n: (n, 0)),
        ],
        out_specs=pl.BlockSpec((_NUM_SAMPLES, bn), lambda n: (0, n)),
        out_shape=jax.ShapeDtypeStruct((_NUM_SAMPLES, p_total), jnp.float32),
    )(sel, x.transpose(1, 2, 3, 0).reshape(p_total, t))

    return out.reshape(_NUM_SAMPLES, c, hh, ww)


# speedup vs baseline: 2.1607x; 1.1038x over previous
import functools

import jax
import jax.numpy as jnp
import numpy as np
from jax.experimental import pallas as pl
from jax.experimental.pallas import tpu as pltpu

_NUM_SAMPLES = 16


def _sample_indices(t: int) -> np.ndarray:
    stop = np.float32(t - 1)
    frac = np.arange(_NUM_SAMPLES - 1, dtype=np.float32) / np.float32(
        _NUM_SAMPLES - 1
    )
    vals = np.concatenate([stop * frac, np.array([stop], np.float32)])
    vals = np.clip(vals, np.float32(0.0), stop)
    return vals.astype(np.int32)


def kernel(x):
    t, c, hh, ww = x.shape
    p_total = c * hh * ww
    src = _sample_indices(t)

    sel_np = np.zeros((_NUM_SAMPLES, t), np.float32)
    sel_np[np.arange(_NUM_SAMPLES), src] = 1.0
    sel = jnp.asarray(sel_np)

    bn = 18816
    grid = p_total // bn
    assert p_total % bn == 0

    def body(sel_ref, x_ref, o_ref):
        o_ref[...] = jax.lax.dot_general(
            sel_ref[...],
            x_ref[...],
            ((( 1,), (1,)), ((), ())),
            preferred_element_type=jnp.float32,
        )

    out = pl.pallas_call(
        body,
        grid=(grid,),
        in_specs=[
            pl.BlockSpec((_NUM_SAMPLES, t), lambda n: (0, 0)),
            pl.BlockSpec((bn, t), lambda n: (n, 0)),
        ],
        out_specs=pl.BlockSpec((_NUM_SAMPLES, bn), lambda n: (0, n)),
        out_shape=jax.ShapeDtypeStruct((_NUM_SAMPLES, p_total), jnp.float32),
    )(sel, x.transpose(1, 2, 3, 0).reshape(p_total, t))

    return out.reshape(_NUM_SAMPLES, c, hh, ww)
